# baseline (speedup 1.0000x reference)
import functools
from itertools import combinations

import jax
import jax.numpy as jnp
import numpy as np
from jax import lax
from jax.experimental import pallas as pl
from jax.experimental.pallas import tpu as pltpu
from jax.experimental.pallas import tpu_sc as plsc

B, F, V, D = 4096, 26, 1000000, 16
FD = F * D
BF = B * F
MLP_W = 256
NUM_PAIRS = F * (F - 1) // 2

NC, NS = 2, 16
NW = NC * NS
CHUNK = BF // NW
IDX_W = 128
NCH = CHUNK // IDX_W

_PAIR_ROWS, _PAIR_COLS = map(np.array, zip(*combinations(range(F), 2)))


def _sc_gather_body(idx_hbm, vtab_hbm, wtab16_hbm, xv_out, xw64_out,
                    idx_v, idxhi_v, rows_v, w64_v, sem_v, sem_w):
    wid = lax.axis_index("s") * NC + lax.axis_index("c")
    base = wid * CHUNK
    pltpu.sync_copy(idx_hbm.at[pl.ds(base, CHUNK)], idx_v)

    def prep(j, carry):
        off = pl.multiple_of(j * 16, 16)
        idxhi_v[pl.ds(off, 16)] = lax.shift_right_logical(idx_v[pl.ds(off, 16)], 4)
        return carry

    lax.fori_loop(0, CHUNK // 16, prep, 0)

    def chunk(j, carry):
        off = pl.multiple_of(j * IDX_W, IDX_W)
        cp_v = pltpu.async_copy(vtab_hbm.at[idx_v.at[pl.ds(off, IDX_W)]],
                                rows_v.at[pl.ds(off, IDX_W)], sem_v)
        cp_w = pltpu.async_copy(wtab16_hbm.at[idxhi_v.at[pl.ds(off, IDX_W)]],
                                w64_v.at[pl.ds(off, IDX_W)], sem_w)
        cp_v.wait()
        cp_w.wait()
        return carry

    lax.fori_loop(0, NCH, chunk, 0)
    pltpu.sync_copy(rows_v, xv_out.at[pl.ds(base, CHUNK)])
    pltpu.sync_copy(w64_v, xw64_out.at[pl.ds(base, CHUNK)])


@functools.cache
def _sc_gather():
    return pl.kernel(
        _sc_gather_body,
        out_type=[
            jax.ShapeDtypeStruct((BF, D), jnp.float32),
            jax.ShapeDtypeStruct((BF, 16), jnp.float32),
        ],
        mesh=plsc.VectorSubcoreMesh(core_axis_name="c", subcore_axis_name="s"),
        compiler_params=pltpu.CompilerParams(use_tc_tiling_on_sc=False),
        scratch_types=[
            pltpu.VMEM((CHUNK,), jnp.int32),
            pltpu.VMEM((CHUNK,), jnp.int32),
            pltpu.VMEM((CHUNK, D), jnp.float32),
            pltpu.VMEM((CHUNK, 16), jnp.float32),
            pltpu.SemaphoreType.DMA,
            pltpu.SemaphoreType.DMA,
        ],
    )


BB = 512
GRID = B // BB


def _tc_body(xv_ref, fv_ref, w64_ref, lane_ref, w0_ref, b0_ref, w1_ref, b1_ref,
             w2_ref, b2_ref, w3t_ref, k_ref, e_ref, sc_ref, out_ref):
    fv = fv_ref[...]
    e_mat = e_ref[...]
    val_exp = jnp.dot(fv, e_mat, preferred_element_type=jnp.float32)
    xv = xv_ref[...] * val_exp
    h = jnp.maximum(jnp.dot(xv, w0_ref[...], preferred_element_type=jnp.float32)
                    + b0_ref[...], 0.0)
    h = jnp.maximum(jnp.dot(h, w1_ref[...], preferred_element_type=jnp.float32)
                    + b1_ref[...], 0.0)
    h = jnp.maximum(jnp.dot(h, w2_ref[...], preferred_element_type=jnp.float32)
                    + b2_ref[...], 0.0)
    deep = jnp.sum(h * w3t_ref[...], axis=1, keepdims=True)
    z = jnp.dot(xv, k_ref[...], preferred_element_type=jnp.float32)
    fm = 0.5 * jnp.sum(xv * z, axis=1, keepdims=True)
    lane_exp = jnp.dot(lane_ref[...], e_mat, preferred_element_type=jnp.float32)
    lanepat = (lax.broadcasted_iota(jnp.int32, (BB, FD), 1) % 16).astype(jnp.float32)
    onehot = (lane_exp == lanepat).astype(jnp.float32)
    lin = jnp.sum(w64_ref[...] * val_exp * onehot, axis=1, keepdims=True)
    out_ref[...] = (lin + fm + deep) * sc_ref[0] + sc_ref[1]


_tc_call = pl.pallas_call(
    _tc_body,
    grid=(GRID,),
    in_specs=[
        pl.BlockSpec((BB, FD), lambda i: (i, 0)),
        pl.BlockSpec((BB, F), lambda i: (i, 0)),
        pl.BlockSpec((BB, FD), lambda i: (i, 0)),
        pl.BlockSpec((BB, F), lambda i: (i, 0)),
        pl.BlockSpec((FD, MLP_W), lambda i: (0, 0)),
        pl.BlockSpec((1, MLP_W), lambda i: (0, 0)),
        pl.BlockSpec((MLP_W, MLP_W), lambda i: (0, 0)),
        pl.BlockSpec((1, MLP_W), lambda i: (0, 0)),
        pl.BlockSpec((MLP_W, MLP_W), lambda i: (0, 0)),
        pl.BlockSpec((1, MLP_W), lambda i: (0, 0)),
        pl.BlockSpec((1, MLP_W), lambda i: (0, 0)),
        pl.BlockSpec((FD, FD), lambda i: (0, 0)),
        pl.BlockSpec((F, FD), lambda i: (0, 0)),
        pl.BlockSpec(memory_space=pltpu.SMEM),
    ],
    out_specs=pl.BlockSpec((BB, 1), lambda i: (i, 0)),
    out_shape=jax.ShapeDtypeStruct((B, 1), jnp.float32),
)

_E = np.kron(np.eye(F, dtype=np.float32), np.ones((1, D), dtype=np.float32))


def kernel(feat_index, feat_value, w_table, v_table, W0, b0, W1, b1, W2, b2,
           W3, b3, mask, bn_gamma, bn_beta, W_out, b_out):
    idx_flat = feat_index.reshape(BF)
    xv_raw, xw64_raw = _sc_gather()(idx_flat, v_table, w_table.reshape(V // 16, 16))
    xv2 = xv_raw.reshape(B, FD)
    w64_2 = xw64_raw.reshape(B, FD)
    lane_f = (feat_index % 16).astype(jnp.float32)

    coef = (mask[0] * bn_gamma) * (1.0 / np.sqrt(1.0 + 1e-3))
    M = jnp.zeros((F, F), jnp.float32).at[_PAIR_ROWS, _PAIR_COLS].set(coef)
    M = M + M.T
    K = jnp.kron(M, jnp.eye(D, dtype=jnp.float32))
    scale = W_out[0, 0]
    shift = b_out[0] + scale * (b3[0] + jnp.sum(mask[0] * bn_beta))
    sc = jnp.stack([scale, shift])

    return _tc_call(
        xv2, feat_value, w64_2, lane_f, W0, b0.reshape(1, MLP_W), W1,
        b1.reshape(1, MLP_W), W2, b2.reshape(1, MLP_W), W3.reshape(1, MLP_W),
        K, _E, sc)
